# baseline (speedup 1.0000x reference)
import functools

import jax
import jax.numpy as jnp
from jax.experimental import pallas as pl
from jax.experimental.pallas import tpu as pltpu

_NUM_CLASSES = 21
_THRESHOLD = 0.5
_V0 = 0.1
_V1 = 0.2
_NEGPOS_RATIO = 3


def _body(conf_ref, loc_ref, pri_ref, tgt_ref, out_l_ref, out_c_ref,
          bits_ref, k_ref, acc_ref, *, B, P, C, O, G):
    g = pl.program_id(0)
    nsteps = B // G

    @pl.when(g == 0)
    def _init():
        acc_ref[0] = 0.0
        acc_ref[1] = 0.0
        acc_ref[2] = 0.0
        out_l_ref[...] = jnp.zeros((1, 1), jnp.float32)
        out_c_ref[...] = jnp.zeros((1, 1), jnp.float32)

    pcx = pri_ref[0:1, :]
    pcy = pri_ref[1:2, :]
    pw = pri_ref[2:3, :]
    ph = pri_ref[3:4, :]
    pfx1 = pcx - pw * 0.5
    pfy1 = pcy - ph * 0.5
    pfx2 = pcx + pw * 0.5
    pfy2 = pcy + ph * 0.5
    area_p = pw * ph

    iota_p = jax.lax.broadcasted_iota(jnp.int32, (G, P), 1)

    best_ov = jnp.full((G, P), -1.0, jnp.float32)
    m_x1 = jnp.zeros((G, P), jnp.float32)
    m_y1 = jnp.zeros((G, P), jnp.float32)
    m_x2 = jnp.zeros((G, P), jnp.float32)
    m_y2 = jnp.zeros((G, P), jnp.float32)
    m_lab = jnp.zeros((G, P), jnp.float32)

    truth_vals = []
    best_prior_idx = []
    for o in range(O):
        tx1 = tgt_ref[:, o, 0:1]
        ty1 = tgt_ref[:, o, 1:2]
        tx2 = tgt_ref[:, o, 2:3]
        ty2 = tgt_ref[:, o, 3:4]
        lab = tgt_ref[:, o, 4:5]
        truth_vals.append((tx1, ty1, tx2, ty2, lab))
        iw = jnp.maximum(jnp.minimum(tx2, pfx2) - jnp.maximum(tx1, pfx1), 0.0)
        ih = jnp.maximum(jnp.minimum(ty2, pfy2) - jnp.maximum(ty1, pfy1), 0.0)
        inter = iw * ih
        ta = (tx2 - tx1) * (ty2 - ty1)
        ov = inter / (ta + area_p - inter)
        mo = jnp.max(ov, axis=1, keepdims=True)
        bpi = jnp.min(jnp.where(ov == mo, iota_p, P), axis=1, keepdims=True)
        best_prior_idx.append(bpi)
        upd = ov > best_ov
        best_ov = jnp.where(upd, ov, best_ov)
        m_x1 = jnp.where(upd, tx1, m_x1)
        m_y1 = jnp.where(upd, ty1, m_y1)
        m_x2 = jnp.where(upd, tx2, m_x2)
        m_y2 = jnp.where(upd, ty2, m_y2)
        m_lab = jnp.where(upd, lab, m_lab)

    for o in range(O):
        tx1, ty1, tx2, ty2, lab = truth_vals[o]
        force = iota_p == best_prior_idx[o]
        best_ov = jnp.where(force, 2.0, best_ov)
        m_x1 = jnp.where(force, tx1, m_x1)
        m_y1 = jnp.where(force, ty1, m_y1)
        m_x2 = jnp.where(force, tx2, m_x2)
        m_y2 = jnp.where(force, ty2, m_y2)
        m_lab = jnp.where(force, lab, m_lab)

    conf_t = jnp.where(best_ov < _THRESHOLD, 0, m_lab.astype(jnp.int32) + 1)
    pos = conf_t > 0
    npos_i = jnp.sum(pos.astype(jnp.int32), axis=1, keepdims=True)
    npos = jnp.sum(npos_i)

    g_cx = ((m_x1 + m_x2) * 0.5 - pcx) / (_V0 * pw)
    g_cy = ((m_y1 + m_y2) * 0.5 - pcy) / (_V0 * ph)
    g_w = jnp.log((m_x2 - m_x1) / pw) / _V1
    g_h = jnp.log((m_y2 - m_y1) / ph) / _V1

    lsum = jnp.float32(0.0)
    for comp, gv in enumerate((g_cx, g_cy, g_w, g_h)):
        d = loc_ref[:, comp, :] - gv
        ad = jnp.abs(d)
        sl1 = jnp.where(ad < 1.0, 0.5 * ad * ad, ad - 0.5)
        lsum = lsum + jnp.sum(jnp.where(pos, sl1, 0.0))

    cb = conf_ref[...]
    mx = jnp.max(cb, axis=1, keepdims=True)
    s = jnp.sum(jnp.exp(cb - mx), axis=1, keepdims=True)
    lse = jnp.log(s[:, 0, :]) + mx[:, 0, :]
    cidx = jax.lax.broadcasted_iota(jnp.int32, (G, C, P), 1)
    logit_t = jnp.sum(jnp.where(cidx == conf_t[:, None, :], cb, 0.0), axis=1)
    nll = lse - logit_t
    lc = jnp.where(pos, 0.0, nll)

    cpos = jnp.sum(jnp.where(pos, nll, 0.0))

    bits_ref[pl.ds(g * G, G), :] = jax.lax.bitcast_convert_type(lc, jnp.int32)
    kvec = jnp.minimum(_NEGPOS_RATIO * npos_i, P - 1)
    k_ref[pl.ds(g * G, G), :] = jnp.broadcast_to(kvec, (G, 128))

    acc_ref[0] = acc_ref[0] + lsum
    acc_ref[1] = acc_ref[1] + cpos
    acc_ref[2] = acc_ref[2] + npos.astype(jnp.float32)

    @pl.when(g == nsteps - 1)
    def _final():
        bits = bits_ref[...]
        kv = k_ref[:, 0:1]

        def it(i, t):
            bit = 30 - i
            cand = t | jnp.left_shift(jnp.int32(1), bit)
            cnt = jnp.sum((bits >= cand).astype(jnp.int32), axis=1,
                          keepdims=True)
            return jnp.where(cnt >= kv, cand, t)

        t = jax.lax.fori_loop(0, 31, it, jnp.zeros((B, 1), jnp.int32))
        gt = bits > t
        cnt_gt = jnp.sum(gt.astype(jnp.int32), axis=1, keepdims=True)
        lcf = jax.lax.bitcast_convert_type(bits, jnp.float32)
        ssel = jnp.sum(jnp.where(gt, lcf, 0.0), axis=1, keepdims=True)
        tf = jax.lax.bitcast_convert_type(t, jnp.float32)
        tf = jnp.where(kv > 0, tf, 0.0)
        rows = ssel + (kv - cnt_gt).astype(jnp.float32) * tf
        cneg = jnp.sum(rows)
        n = jnp.maximum(acc_ref[2], 1.0)
        out_l_ref[...] = jnp.full((1, 1), acc_ref[0] / n, jnp.float32)
        out_c_ref[...] = jnp.full((1, 1), (acc_ref[1] + cneg) / n, jnp.float32)


def _run(loc_t, conf_t, priors_t, targets, interpret=False):
    B, C, P = conf_t.shape
    O = targets.shape[1]
    G = 8
    body = functools.partial(_body, B=B, P=P, C=C, O=O, G=G)
    out = pl.pallas_call(
        body,
        grid=(B // G,),
        in_specs=[
            pl.BlockSpec((G, C, P), lambda g: (g, 0, 0)),
            pl.BlockSpec((G, 4, P), lambda g: (g, 0, 0)),
            pl.BlockSpec((4, P), lambda g: (0, 0)),
            pl.BlockSpec((G, O, 5), lambda g: (g, 0, 0)),
        ],
        out_specs=[
            pl.BlockSpec((1, 1), lambda g: (0, 0)),
            pl.BlockSpec((1, 1), lambda g: (0, 0)),
        ],
        out_shape=[
            jax.ShapeDtypeStruct((1, 1), jnp.float32),
            jax.ShapeDtypeStruct((1, 1), jnp.float32),
        ],
        scratch_shapes=[
            pltpu.VMEM((B, P), jnp.int32),
            pltpu.VMEM((B, 128), jnp.int32),
            pltpu.SMEM((4,), jnp.float32),
        ],
        interpret=interpret,
    )(conf_t, loc_t, priors_t, targets)
    return out


def kernel(loc_data, conf_data, priors, targets):
    conf_t = jnp.transpose(conf_data, (0, 2, 1))
    loc_t = jnp.transpose(loc_data, (0, 2, 1))
    priors_t = priors.T
    out_l, out_c = _run(loc_t, conf_t, priors_t, targets)
    return out_l[0, 0], out_c[0, 0]


# speedup vs baseline: 1.0577x; 1.0577x over previous
import functools

import jax
import jax.numpy as jnp
from jax.experimental import pallas as pl
from jax.experimental.pallas import tpu as pltpu

_NUM_CLASSES = 21
_THRESHOLD = 0.5
_V0 = 0.1
_V1 = 0.2
_NEGPOS_RATIO = 3


def _body(conf_ref, loc_ref, pri_ref, tgt_ref, out_l_ref, out_c_ref,
          bits_ref, k_ref, acc_ref, *, B, P, C, O, G):
    g = pl.program_id(0)
    nsteps = B // G

    @pl.when(g == 0)
    def _init():
        acc_ref[0] = 0.0
        acc_ref[1] = 0.0
        acc_ref[2] = 0.0
        out_l_ref[...] = jnp.zeros((1, 1), jnp.float32)
        out_c_ref[...] = jnp.zeros((1, 1), jnp.float32)

    pcx = pri_ref[0:1, :]
    pcy = pri_ref[1:2, :]
    pw = pri_ref[2:3, :]
    ph = pri_ref[3:4, :]
    pfx1 = pcx - pw * 0.5
    pfy1 = pcy - ph * 0.5
    pfx2 = pcx + pw * 0.5
    pfy2 = pcy + ph * 0.5
    area_p = pw * ph

    iota_p = jax.lax.broadcasted_iota(jnp.int32, (G, P), 1)

    best_ov = jnp.full((G, P), -1.0, jnp.float32)
    m_x1 = jnp.zeros((G, P), jnp.float32)
    m_y1 = jnp.zeros((G, P), jnp.float32)
    m_x2 = jnp.zeros((G, P), jnp.float32)
    m_y2 = jnp.zeros((G, P), jnp.float32)
    m_lab = jnp.zeros((G, P), jnp.float32)

    truth_vals = []
    best_prior_idx = []
    for o in range(O):
        tx1 = tgt_ref[:, o, 0:1]
        ty1 = tgt_ref[:, o, 1:2]
        tx2 = tgt_ref[:, o, 2:3]
        ty2 = tgt_ref[:, o, 3:4]
        lab = tgt_ref[:, o, 4:5]
        truth_vals.append((tx1, ty1, tx2, ty2, lab))
        iw = jnp.maximum(jnp.minimum(tx2, pfx2) - jnp.maximum(tx1, pfx1), 0.0)
        ih = jnp.maximum(jnp.minimum(ty2, pfy2) - jnp.maximum(ty1, pfy1), 0.0)
        inter = iw * ih
        ta = (tx2 - tx1) * (ty2 - ty1)
        ov = inter / (ta + area_p - inter)
        mo = jnp.max(ov, axis=1, keepdims=True)
        bpi = jnp.min(jnp.where(ov == mo, iota_p, P), axis=1, keepdims=True)
        best_prior_idx.append(bpi)
        upd = ov > best_ov
        best_ov = jnp.where(upd, ov, best_ov)
        m_x1 = jnp.where(upd, tx1, m_x1)
        m_y1 = jnp.where(upd, ty1, m_y1)
        m_x2 = jnp.where(upd, tx2, m_x2)
        m_y2 = jnp.where(upd, ty2, m_y2)
        m_lab = jnp.where(upd, lab, m_lab)

    for o in range(O):
        tx1, ty1, tx2, ty2, lab = truth_vals[o]
        force = iota_p == best_prior_idx[o]
        best_ov = jnp.where(force, 2.0, best_ov)
        m_x1 = jnp.where(force, tx1, m_x1)
        m_y1 = jnp.where(force, ty1, m_y1)
        m_x2 = jnp.where(force, tx2, m_x2)
        m_y2 = jnp.where(force, ty2, m_y2)
        m_lab = jnp.where(force, lab, m_lab)

    conf_t = jnp.where(best_ov < _THRESHOLD, 0, m_lab.astype(jnp.int32) + 1)
    pos = conf_t > 0
    npos_i = jnp.sum(pos.astype(jnp.int32), axis=1, keepdims=True)
    npos = jnp.sum(npos_i)

    g_cx = ((m_x1 + m_x2) * 0.5 - pcx) / (_V0 * pw)
    g_cy = ((m_y1 + m_y2) * 0.5 - pcy) / (_V0 * ph)
    g_w = jnp.log((m_x2 - m_x1) / pw) / _V1
    g_h = jnp.log((m_y2 - m_y1) / ph) / _V1

    lsum = jnp.float32(0.0)
    for comp, gv in enumerate((g_cx, g_cy, g_w, g_h)):
        d = loc_ref[:, comp, :] - gv
        ad = jnp.abs(d)
        sl1 = jnp.where(ad < 1.0, 0.5 * ad * ad, ad - 0.5)
        lsum = lsum + jnp.sum(jnp.where(pos, sl1, 0.0))

    cb = conf_ref[...]
    mx = jnp.max(cb, axis=1, keepdims=True)
    s = jnp.sum(jnp.exp(cb - mx), axis=1, keepdims=True)
    lse = jnp.log(s[:, 0, :]) + mx[:, 0, :]
    cidx = jax.lax.broadcasted_iota(jnp.int32, (G, C, P), 1)
    logit_t = jnp.sum(jnp.where(cidx == conf_t[:, None, :], cb, 0.0), axis=1)
    nll = lse - logit_t
    lc = jnp.where(pos, 0.0, nll)

    cpos = jnp.sum(jnp.where(pos, nll, 0.0))

    bits_ref[pl.ds(g * G, G), :] = jax.lax.bitcast_convert_type(lc, jnp.int32)
    kvec = jnp.minimum(_NEGPOS_RATIO * npos_i, P - 1)
    k_ref[pl.ds(g * G, G), :] = jnp.broadcast_to(kvec, (G, 128))

    acc_ref[0] = acc_ref[0] + lsum
    acc_ref[1] = acc_ref[1] + cpos
    acc_ref[2] = acc_ref[2] + npos.astype(jnp.float32)

    @pl.when(g == nsteps - 1)
    def _final():
        bits = bits_ref[...]
        kv = k_ref[:, 0:1]

        def it(i, t):
            bit = 30 - i
            cand = t | jnp.left_shift(jnp.int32(1), bit)
            cnt = jnp.sum((bits >= cand).astype(jnp.int32), axis=1,
                          keepdims=True)
            return jnp.where(cnt >= kv, cand, t)

        t = jax.lax.fori_loop(0, 31, it, jnp.zeros((B, 1), jnp.int32))
        gt = bits > t
        cnt_gt = jnp.sum(gt.astype(jnp.int32), axis=1, keepdims=True)
        lcf = jax.lax.bitcast_convert_type(bits, jnp.float32)
        ssel = jnp.sum(jnp.where(gt, lcf, 0.0), axis=1, keepdims=True)
        tf = jax.lax.bitcast_convert_type(t, jnp.float32)
        tf = jnp.where(kv > 0, tf, 0.0)
        rows = ssel + (kv - cnt_gt).astype(jnp.float32) * tf
        cneg = jnp.sum(rows)
        n = jnp.maximum(acc_ref[2], 1.0)
        out_l_ref[...] = jnp.full((1, 1), acc_ref[0] / n, jnp.float32)
        out_c_ref[...] = jnp.full((1, 1), (acc_ref[1] + cneg) / n, jnp.float32)


def _run(loc_t, conf_t, priors_t, targets, interpret=False):
    B, C, P = conf_t.shape
    O = targets.shape[1]
    G = 1
    body = functools.partial(_body, B=B, P=P, C=C, O=O, G=G)
    out = pl.pallas_call(
        body,
        grid=(B // G,),
        in_specs=[
            pl.BlockSpec((G, C, P), lambda g: (g, 0, 0)),
            pl.BlockSpec((G, 4, P), lambda g: (g, 0, 0)),
            pl.BlockSpec((4, P), lambda g: (0, 0)),
            pl.BlockSpec((G, O, 5), lambda g: (g, 0, 0)),
        ],
        out_specs=[
            pl.BlockSpec((1, 1), lambda g: (0, 0)),
            pl.BlockSpec((1, 1), lambda g: (0, 0)),
        ],
        out_shape=[
            jax.ShapeDtypeStruct((1, 1), jnp.float32),
            jax.ShapeDtypeStruct((1, 1), jnp.float32),
        ],
        scratch_shapes=[
            pltpu.VMEM((B, P), jnp.int32),
            pltpu.VMEM((B, 128), jnp.int32),
            pltpu.SMEM((4,), jnp.float32),
        ],
        interpret=interpret,
    )(conf_t, loc_t, priors_t, targets)
    return out


def kernel(loc_data, conf_data, priors, targets):
    conf_t = jnp.transpose(conf_data, (0, 2, 1))
    loc_t = jnp.transpose(loc_data, (0, 2, 1))
    priors_t = priors.T
    out_l, out_c = _run(loc_t, conf_t, priors_t, targets)
    return out_l[0, 0], out_c[0, 0]


# speedup vs baseline: 1.1444x; 1.0821x over previous
import functools

import jax
import jax.numpy as jnp
from jax.experimental import pallas as pl
from jax.experimental.pallas import tpu as pltpu

_NUM_CLASSES = 21
_THRESHOLD = 0.5
_V0 = 0.1
_V1 = 0.2
_NEGPOS_RATIO = 3


def _body(conf_ref, loc_ref, pri_ref, tgt_ref, out_l_ref, out_c_ref,
          bits_ref, k_ref, acc_ref, *, B, P, C, O, G):
    g = pl.program_id(0)
    nsteps = B // G

    @pl.when(g == 0)
    def _init():
        acc_ref[0] = 0.0
        acc_ref[1] = 0.0
        acc_ref[2] = 0.0
        out_l_ref[...] = jnp.zeros((1, 1), jnp.float32)
        out_c_ref[...] = jnp.zeros((1, 1), jnp.float32)

    pcx = pri_ref[0:1, :]
    pcy = pri_ref[1:2, :]
    pw = pri_ref[2:3, :]
    ph = pri_ref[3:4, :]
    pfx1 = pcx - pw * 0.5
    pfy1 = pcy - ph * 0.5
    pfx2 = pcx + pw * 0.5
    pfy2 = pcy + ph * 0.5
    area_p = pw * ph

    iota_p = jax.lax.broadcasted_iota(jnp.int32, (G, P), 1)

    best_ov = jnp.full((G, P), -1.0, jnp.float32)
    m_x1 = jnp.zeros((G, P), jnp.float32)
    m_y1 = jnp.zeros((G, P), jnp.float32)
    m_x2 = jnp.zeros((G, P), jnp.float32)
    m_y2 = jnp.zeros((G, P), jnp.float32)
    m_lab = jnp.zeros((G, P), jnp.float32)

    for o in range(O):
        tx1 = tgt_ref[:, o, 0:1]
        ty1 = tgt_ref[:, o, 1:2]
        tx2 = tgt_ref[:, o, 2:3]
        ty2 = tgt_ref[:, o, 3:4]
        lab = tgt_ref[:, o, 4:5]
        iw = jnp.maximum(jnp.minimum(tx2, pfx2) - jnp.maximum(tx1, pfx1), 0.0)
        ih = jnp.maximum(jnp.minimum(ty2, pfy2) - jnp.maximum(ty1, pfy1), 0.0)
        inter = iw * ih
        ta = (tx2 - tx1) * (ty2 - ty1)
        ov = inter / (ta + area_p - inter)
        mo = jnp.max(ov, axis=1, keepdims=True)
        force = ov == mo
        force = jnp.logical_and(
            force,
            iota_p == jnp.min(jnp.where(force, iota_p, P), axis=1,
                              keepdims=True))
        upd = ov > best_ov
        cond = jnp.logical_or(upd, force)
        best_ov = jnp.where(force, 2.0, jnp.where(upd, ov, best_ov))
        m_x1 = jnp.where(cond, tx1, m_x1)
        m_y1 = jnp.where(cond, ty1, m_y1)
        m_x2 = jnp.where(cond, tx2, m_x2)
        m_y2 = jnp.where(cond, ty2, m_y2)
        m_lab = jnp.where(cond, lab, m_lab)

    conf_t = jnp.where(best_ov < _THRESHOLD, 0, m_lab.astype(jnp.int32) + 1)
    pos = conf_t > 0
    npos_i = jnp.sum(pos.astype(jnp.int32), axis=1, keepdims=True)
    npos = jnp.sum(npos_i)

    g_cx = ((m_x1 + m_x2) * 0.5 - pcx) / (_V0 * pw)
    g_cy = ((m_y1 + m_y2) * 0.5 - pcy) / (_V0 * ph)
    g_w = jnp.log((m_x2 - m_x1) / pw) / _V1
    g_h = jnp.log((m_y2 - m_y1) / ph) / _V1

    lsum = jnp.float32(0.0)
    for comp, gv in enumerate((g_cx, g_cy, g_w, g_h)):
        d = loc_ref[:, comp, :] - gv
        ad = jnp.abs(d)
        sl1 = jnp.where(ad < 1.0, 0.5 * ad * ad, ad - 0.5)
        lsum = lsum + jnp.sum(jnp.where(pos, sl1, 0.0))

    cb = conf_ref[...]
    mx = jnp.max(cb, axis=1, keepdims=True)
    s = jnp.sum(jnp.exp(cb - mx), axis=1, keepdims=True)
    lse = jnp.log(s[:, 0, :]) + mx[:, 0, :]
    cidx = jax.lax.broadcasted_iota(jnp.int32, (G, C, P), 1)
    logit_t = jnp.sum(jnp.where(cidx == conf_t[:, None, :], cb, 0.0), axis=1)
    nll = lse - logit_t
    lc = jnp.where(pos, 0.0, nll)

    cpos = jnp.sum(jnp.where(pos, nll, 0.0))

    bits_ref[pl.ds(g * G, G), :] = jax.lax.bitcast_convert_type(lc, jnp.int32)
    kvec = jnp.minimum(_NEGPOS_RATIO * npos_i, P - 1)
    k_ref[pl.ds(g * G, G), :] = jnp.broadcast_to(kvec, (G, 128))

    acc_ref[0] = acc_ref[0] + lsum
    acc_ref[1] = acc_ref[1] + cpos
    acc_ref[2] = acc_ref[2] + npos.astype(jnp.float32)

    @pl.when(g == nsteps - 1)
    def _final():
        bits = bits_ref[...]
        kv = k_ref[:, 0:1]

        def it(i, t):
            bit = 30 - i
            cand = t | jnp.left_shift(jnp.int32(1), bit)
            cnt = jnp.sum((bits >= cand).astype(jnp.int32), axis=1,
                          keepdims=True)
            return jnp.where(cnt >= kv, cand, t)

        t = jax.lax.fori_loop(0, 31, it, jnp.zeros((B, 1), jnp.int32))
        gt = bits > t
        cnt_gt = jnp.sum(gt.astype(jnp.int32), axis=1, keepdims=True)
        lcf = jax.lax.bitcast_convert_type(bits, jnp.float32)
        ssel = jnp.sum(jnp.where(gt, lcf, 0.0), axis=1, keepdims=True)
        tf = jax.lax.bitcast_convert_type(t, jnp.float32)
        tf = jnp.where(kv > 0, tf, 0.0)
        rows = ssel + (kv - cnt_gt).astype(jnp.float32) * tf
        cneg = jnp.sum(rows)
        n = jnp.maximum(acc_ref[2], 1.0)
        out_l_ref[...] = jnp.full((1, 1), acc_ref[0] / n, jnp.float32)
        out_c_ref[...] = jnp.full((1, 1), (acc_ref[1] + cneg) / n, jnp.float32)


def _run(loc_t, conf_t, priors_t, targets, interpret=False):
    B, C, P = conf_t.shape
    O = targets.shape[1]
    G = 1
    body = functools.partial(_body, B=B, P=P, C=C, O=O, G=G)
    out = pl.pallas_call(
        body,
        grid=(B // G,),
        in_specs=[
            pl.BlockSpec((G, C, P), lambda g: (g, 0, 0)),
            pl.BlockSpec((G, 4, P), lambda g: (g, 0, 0)),
            pl.BlockSpec((4, P), lambda g: (0, 0)),
            pl.BlockSpec((G, O, 5), lambda g: (g, 0, 0)),
        ],
        out_specs=[
            pl.BlockSpec((1, 1), lambda g: (0, 0)),
            pl.BlockSpec((1, 1), lambda g: (0, 0)),
        ],
        out_shape=[
            jax.ShapeDtypeStruct((1, 1), jnp.float32),
            jax.ShapeDtypeStruct((1, 1), jnp.float32),
        ],
        scratch_shapes=[
            pltpu.VMEM((B, P), jnp.int32),
            pltpu.VMEM((B, 128), jnp.int32),
            pltpu.SMEM((4,), jnp.float32),
        ],
        interpret=interpret,
    )(conf_t, loc_t, priors_t, targets)
    return out


def kernel(loc_data, conf_data, priors, targets):
    conf_t = jnp.transpose(conf_data, (0, 2, 1))
    loc_t = jnp.transpose(loc_data, (0, 2, 1))
    priors_t = priors.T
    out_l, out_c = _run(loc_t, conf_t, priors_t, targets)
    return out_l[0, 0], out_c[0, 0]


# speedup vs baseline: 1.3616x; 1.1898x over previous
import functools

import jax
import jax.numpy as jnp
from jax.experimental import pallas as pl
from jax.experimental.pallas import tpu as pltpu

_NUM_CLASSES = 21
_THRESHOLD = 0.5
_V0 = 0.1
_V1 = 0.2
_NEGPOS_RATIO = 3
_LANES = 128


def _body(conf_ref, loc_ref, pri_ref, tgt_ref, out_l_ref, out_c_ref,
          bits_ref, k_ref, acc_ref, *, B, P, C, O, R):
    g = pl.program_id(0)

    @pl.when(g == 0)
    def _init():
        acc_ref[0] = 0.0
        acc_ref[1] = 0.0
        acc_ref[2] = 0.0
        out_l_ref[...] = jnp.zeros((1, 1), jnp.float32)
        out_c_ref[...] = jnp.zeros((1, 1), jnp.float32)

    pcx = pri_ref[0]
    pcy = pri_ref[1]
    pw = pri_ref[2]
    ph = pri_ref[3]
    pfx1 = pcx - pw * 0.5
    pfy1 = pcy - ph * 0.5
    pfx2 = pcx + pw * 0.5
    pfy2 = pcy + ph * 0.5
    area_p = pw * ph

    idx_p = (jax.lax.broadcasted_iota(jnp.int32, (R, _LANES), 0) * _LANES
             + jax.lax.broadcasted_iota(jnp.int32, (R, _LANES), 1))

    best_ov = jnp.full((R, _LANES), -1.0, jnp.float32)
    m_x1 = jnp.zeros((R, _LANES), jnp.float32)
    m_y1 = jnp.zeros((R, _LANES), jnp.float32)
    m_x2 = jnp.zeros((R, _LANES), jnp.float32)
    m_y2 = jnp.zeros((R, _LANES), jnp.float32)
    m_lab = jnp.zeros((R, _LANES), jnp.float32)

    for o in range(O):
        tx1 = tgt_ref[:, o, 0:1]
        ty1 = tgt_ref[:, o, 1:2]
        tx2 = tgt_ref[:, o, 2:3]
        ty2 = tgt_ref[:, o, 3:4]
        lab = tgt_ref[:, o, 4:5]
        iw = jnp.maximum(jnp.minimum(tx2, pfx2) - jnp.maximum(tx1, pfx1), 0.0)
        ih = jnp.maximum(jnp.minimum(ty2, pfy2) - jnp.maximum(ty1, pfy1), 0.0)
        inter = iw * ih
        ta = (tx2 - tx1) * (ty2 - ty1)
        ov = inter / (ta + area_p - inter)
        mo = jnp.max(ov)
        atmax = ov == mo
        force = jnp.logical_and(
            atmax, idx_p == jnp.min(jnp.where(atmax, idx_p, P)))
        upd = ov > best_ov
        cond = jnp.logical_or(upd, force)
        best_ov = jnp.where(force, 2.0, jnp.where(upd, ov, best_ov))
        m_x1 = jnp.where(cond, tx1, m_x1)
        m_y1 = jnp.where(cond, ty1, m_y1)
        m_x2 = jnp.where(cond, tx2, m_x2)
        m_y2 = jnp.where(cond, ty2, m_y2)
        m_lab = jnp.where(cond, lab, m_lab)

    conf_t = jnp.where(best_ov < _THRESHOLD, 0, m_lab.astype(jnp.int32) + 1)
    pos = conf_t > 0
    npos = jnp.sum(pos.astype(jnp.int32))

    g_cx = ((m_x1 + m_x2) * 0.5 - pcx) / (_V0 * pw)
    g_cy = ((m_y1 + m_y2) * 0.5 - pcy) / (_V0 * ph)
    g_w = jnp.log((m_x2 - m_x1) / pw) / _V1
    g_h = jnp.log((m_y2 - m_y1) / ph) / _V1

    lsum = jnp.float32(0.0)
    for comp, gv in enumerate((g_cx, g_cy, g_w, g_h)):
        d = loc_ref[0, comp] - gv
        ad = jnp.abs(d)
        sl1 = jnp.where(ad < 1.0, 0.5 * ad * ad, ad - 0.5)
        lsum = lsum + jnp.sum(jnp.where(pos, sl1, 0.0))

    cb = conf_ref[0]
    mx = jnp.max(cb, axis=0)
    s = jnp.sum(jnp.exp(cb - mx[None]), axis=0)
    lse = jnp.log(s) + mx
    cidx = jax.lax.broadcasted_iota(jnp.int32, (C, R, _LANES), 0)
    logit_t = jnp.sum(jnp.where(cidx == conf_t[None], cb, 0.0), axis=0)
    nll = lse - logit_t
    lc = jnp.where(jnp.logical_or(pos, idx_p >= P), 0.0, nll)

    cpos = jnp.sum(jnp.where(pos, nll, 0.0))

    bits_ref[pl.ds(g, 1)] = jax.lax.bitcast_convert_type(lc, jnp.int32)[None]
    kvec = jnp.minimum(_NEGPOS_RATIO * npos, P - 1)
    k_ref[pl.ds(g, 1), :] = jnp.broadcast_to(kvec, (1, _LANES))

    acc_ref[0] = acc_ref[0] + lsum
    acc_ref[1] = acc_ref[1] + cpos
    acc_ref[2] = acc_ref[2] + npos.astype(jnp.float32)

    @pl.when(g == B - 1)
    def _final():
        bits = bits_ref[...]
        kv = k_ref[:, 0:1].reshape(B, 1, 1)

        def it(i, t):
            bit = 30 - i
            cand = t | jnp.left_shift(jnp.int32(1), bit)
            cnt = jnp.sum((bits >= cand).astype(jnp.int32), axis=(1, 2),
                          keepdims=True)
            return jnp.where(cnt >= kv, cand, t)

        t = jax.lax.fori_loop(0, 31, it, jnp.zeros((B, 1, 1), jnp.int32))
        gt = bits > t
        cnt_gt = jnp.sum(gt.astype(jnp.int32), axis=(1, 2), keepdims=True)
        lcf = jax.lax.bitcast_convert_type(bits, jnp.float32)
        ssel = jnp.sum(jnp.where(gt, lcf, 0.0), axis=(1, 2), keepdims=True)
        tf = jax.lax.bitcast_convert_type(t, jnp.float32)
        tf = jnp.where(kv > 0, tf, 0.0)
        rows = ssel + (kv - cnt_gt).astype(jnp.float32) * tf
        cneg = jnp.sum(rows)
        n = jnp.maximum(acc_ref[2], 1.0)
        out_l_ref[...] = jnp.full((1, 1), acc_ref[0] / n, jnp.float32)
        out_c_ref[...] = jnp.full((1, 1), (acc_ref[1] + cneg) / n, jnp.float32)


def _run(loc_t, conf_t, priors_t, targets, P, interpret=False):
    B, C, R, L = conf_t.shape
    O = targets.shape[1]
    body = functools.partial(_body, B=B, P=P, C=C, O=O, R=R)
    out = pl.pallas_call(
        body,
        grid=(B,),
        in_specs=[
            pl.BlockSpec((1, C, R, L), lambda g: (g, 0, 0, 0)),
            pl.BlockSpec((1, 4, R, L), lambda g: (g, 0, 0, 0)),
            pl.BlockSpec((4, R, L), lambda g: (0, 0, 0)),
            pl.BlockSpec((1, O, 5), lambda g: (g, 0, 0)),
        ],
        out_specs=[
            pl.BlockSpec((1, 1), lambda g: (0, 0)),
            pl.BlockSpec((1, 1), lambda g: (0, 0)),
        ],
        out_shape=[
            jax.ShapeDtypeStruct((1, 1), jnp.float32),
            jax.ShapeDtypeStruct((1, 1), jnp.float32),
        ],
        scratch_shapes=[
            pltpu.VMEM((B, R, L), jnp.int32),
            pltpu.VMEM((B, L), jnp.int32),
            pltpu.SMEM((4,), jnp.float32),
        ],
        interpret=interpret,
    )(conf_t, loc_t, priors_t, targets)
    return out


def _pack(x, pad):
    x = jnp.transpose(x, (0, 2, 1))
    x = jnp.pad(x, ((0, 0), (0, 0), (0, pad)))
    return x.reshape(x.shape[0], x.shape[1], -1, _LANES)


def kernel(loc_data, conf_data, priors, targets):
    P = conf_data.shape[1]
    pad = (-P) % _LANES
    conf_t = _pack(conf_data, pad)
    loc_t = _pack(loc_data, pad)
    priors_t = jnp.pad(priors.T, ((0, 0), (0, pad))).reshape(4, -1, _LANES)
    out_l, out_c = _run(loc_t, conf_t, priors_t, targets, P)
    return out_l[0, 0], out_c[0, 0]
